# baseline (speedup 1.0000x reference)
import functools

import jax
import jax.numpy as jnp
from jax import lax
from jax.experimental import pallas as pl
from jax.experimental.pallas import tpu as pltpu
from jax.experimental.pallas import tpu_sc as plsc

_B = 128
_V = 100000
_K = 50
_BLK = 256
_NB = -(-_V // _BLK)
_NSB = -(-_NB // 16)
_NBP = _NSB * 16
_VP = _NB * _BLK
_KP = 64
_NEG = float("-inf")


def _splat_f(x):
    return jnp.full((16,), x, jnp.float32)


def _splat_i(x):
    return jnp.full((16,), x, jnp.int32)


def _tree_max(vs):
    vs = list(vs)
    while len(vs) > 1:
        vs = [jnp.maximum(vs[i], vs[i + 1]) for i in range(0, len(vs) - 1, 2)] \
            + ([vs[-1]] if len(vs) % 2 else [])
    return vs[0]


def _tree_min(vs):
    vs = list(vs)
    while len(vs) > 1:
        vs = [jnp.minimum(vs[i], vs[i + 1]) for i in range(0, len(vs) - 1, 2)] \
            + ([vs[-1]] if len(vs) % 2 else [])
    return vs[0]


def _ffs(mask):
    f = plsc.all_reduce_ffs(mask)
    if f.ndim == 0:
        f = jnp.full((16,), f, jnp.int32)
    return f


def _build_topk():
    info = plsc.get_sparse_core_info()
    nc, ns = info.num_cores, info.num_subcores
    nw = nc * ns
    rows_per = _B // nw
    mesh = plsc.VectorSubcoreMesh(core_axis_name="c", subcore_axis_name="s")

    @functools.partial(
        pl.kernel,
        out_type=[
            jax.ShapeDtypeStruct((_B * _KP,), jnp.float32),
            jax.ShapeDtypeStruct((_B * _KP,), jnp.int32),
        ],
        mesh=mesh,
        scratch_types=[
            pltpu.VMEM((_VP,), jnp.float32),
            pltpu.VMEM((_NBP * 16,), jnp.float32),
            pltpu.VMEM((_NSB * 16,), jnp.float32),
            pltpu.VMEM((_KP,), jnp.float32),
            pltpu.VMEM((_KP,), jnp.int32),
            pltpu.SemaphoreType.DMA,
            pltpu.SemaphoreType.DMA,
            pltpu.SemaphoreType.DMA,
            pltpu.SemaphoreType.DMA,
        ],
        compiler_params=pltpu.CompilerParams(needs_layout_passes=False),
    )
    def topk(logits_hbm, vals_hbm, idx_hbm, row_v, lm_v, l2_v, ov_v, oi_v,
             sem0, sem1, sem2, sem3):
        wid = lax.axis_index("s") * nc + lax.axis_index("c")
        lanes = lax.iota(jnp.int32, 16)
        neg = _splat_f(_NEG)
        bigi = _splat_i(1 << 30)

        for t in range(_NB * 16, _NBP * 16, 16):
            lm_v[pl.ds(t, 16)] = neg

        def row_body(r, carry):
            row = wid * rows_per + r
            base = row * _V
            qb = 98
            q = qb * _BLK
            sems = (sem0, sem1, sem2, sem3)
            copies = []
            for i in range(4):
                lo = i * q
                hi = min((i + 1) * q, _V)
                copies.append(pltpu.async_copy(
                    logits_hbm.at[pl.ds(base + lo, hi - lo)],
                    row_v.at[pl.ds(lo, hi - lo)], sems[i]))

            def _blk(j):
                lm_v[pl.ds(j * 16, 16)] = _tree_max(
                    [row_v[pl.ds(j * _BLK + ii * 16, 16)] for ii in range(16)])

            for i in range(4):
                copies[i].wait()
                if i == 3:
                    for t in range(_V, _VP, 16):
                        row_v[pl.ds(t, 16)] = neg
                hi = _NB if i == 3 else qb * (i + 1)
                plsc.parallel_loop(qb * i, hi, unroll=8)(_blk)

            def sb_body(s, c):
                l2_v[pl.ds(s * 16, 16)] = _tree_max(
                    [lm_v[pl.ds(s * 256 + jj * 16, 16)] for jj in range(16)])
                return c

            lax.fori_loop(0, _NSB, sb_body, 0)

            for t in range(0, _KP, 16):
                ov_v[pl.ds(t, 16)] = neg
                oi_v[pl.ds(t, 16)] = _splat_i(0)

            def k_body(k, c):
                g = _tree_max([l2_v[pl.ds(s * 16, 16)] for s in range(_NSB)])
                m = jnp.max(g)
                mv = jnp.full((16,), m, jnp.float32)

                sv = []
                for s in range(_NSB):
                    f = _ffs(l2_v[pl.ds(s * 16, 16)] == mv)
                    sv.append(jnp.where(f < 16, _splat_i(s), bigi))
                sstar = jnp.min(_tree_min(sv))

                bv = []
                for jj in range(16):
                    f = _ffs(lm_v[pl.ds(sstar * 256 + jj * 16, 16)] == mv)
                    bv.append(jnp.where(f < 16, _splat_i(jj), bigi))
                bstar = sstar * 16 + jnp.min(_tree_min(bv))

                pv = []
                for ii in range(16):
                    f = _ffs(row_v[pl.ds(bstar * _BLK + ii * 16, 16)] == mv)
                    pv.append(jnp.where(f < 16, f + ii * 16, bigi))
                pos = jnp.min(_tree_min(pv))
                flat = bstar * _BLK + pos

                q = (k // 16) * 16
                rl = k % 16
                ov_v[pl.ds(q, 16)] = jnp.where(lanes == rl, mv,
                                               ov_v[pl.ds(q, 16)])
                oi_v[pl.ds(q, 16)] = jnp.where(lanes == rl,
                                               jnp.full((16,), flat, jnp.int32),
                                               oi_v[pl.ds(q, 16)])

                base = bstar * _BLK + (pos // 16) * 16
                row_v[pl.ds(base, 16)] = jnp.where(lanes == pos % 16, neg,
                                                   row_v[pl.ds(base, 16)])

                lm_v[pl.ds(bstar * 16, 16)] = _tree_max(
                    [row_v[pl.ds(bstar * _BLK + ii * 16, 16)]
                     for ii in range(16)])
                l2_v[pl.ds(sstar * 16, 16)] = _tree_max(
                    [lm_v[pl.ds(sstar * 256 + jj * 16, 16)]
                     for jj in range(16)])
                return c

            lax.fori_loop(0, _K, k_body, 0)

            pltpu.sync_copy(ov_v, vals_hbm.at[pl.ds(row * _KP, _KP)])
            pltpu.sync_copy(oi_v, idx_hbm.at[pl.ds(row * _KP, _KP)])
            return carry

        lax.fori_loop(0, rows_per, row_body, 0)

    return topk


def _sample_body(vals_ref, idx_ref, u_ref, out_ref):
    v = vals_ref[...]
    lane = lax.broadcasted_iota(jnp.int32, (_B, _KP), 1)
    valid = lane < _K
    vm = jnp.where(valid, v, -jnp.inf)
    rmax = jnp.max(vm, axis=1, keepdims=True)
    e = jnp.where(valid, jnp.exp(vm - rmax), 0.0)
    p = e / jnp.sum(e, axis=1, keepdims=True)
    g = -jnp.log(-jnp.log(u_ref[...]))
    score = jnp.where(valid, jnp.log(p + 1e-12) + g, -jnp.inf)
    smax = jnp.max(score, axis=1, keepdims=True)
    choice = jnp.min(jnp.where(score == smax, lane, _KP), axis=1, keepdims=True)
    token = jnp.sum(jnp.where(lane == choice, idx_ref[...], 0), axis=1,
                    keepdims=True)
    out_ref[...] = token


def kernel(logits, u):
    vals_f, idx_f = _build_topk()(logits.reshape(-1))
    vals = vals_f.reshape(_B, _KP)
    idx = idx_f.reshape(_B, _KP)
    up = jnp.pad(u, ((0, 0), (0, _KP - _K)), constant_values=0.5)
    token = pl.pallas_call(
        _sample_body,
        out_shape=jax.ShapeDtypeStruct((_B, 1), jnp.int32),
    )(vals, idx, up)
    return token.reshape(_B)


# speedup vs baseline: 1.0168x; 1.0168x over previous
import functools

import jax
import jax.numpy as jnp
from jax import lax
from jax.experimental import pallas as pl
from jax.experimental.pallas import tpu as pltpu
from jax.experimental.pallas import tpu_sc as plsc

_B = 128
_V = 100000
_K = 50
_BLK = 256
_NB = -(-_V // _BLK)
_NSB = -(-_NB // 16)
_NBP = _NSB * 16
_VP = _NB * _BLK
_KP = 64
_NEG = float("-inf")


def _splat_f(x):
    return jnp.full((16,), x, jnp.float32)


def _splat_i(x):
    return jnp.full((16,), x, jnp.int32)


def _tree_max(vs):
    vs = list(vs)
    while len(vs) > 1:
        vs = [jnp.maximum(vs[i], vs[i + 1]) for i in range(0, len(vs) - 1, 2)] \
            + ([vs[-1]] if len(vs) % 2 else [])
    return vs[0]


def _tree_min(vs):
    vs = list(vs)
    while len(vs) > 1:
        vs = [jnp.minimum(vs[i], vs[i + 1]) for i in range(0, len(vs) - 1, 2)] \
            + ([vs[-1]] if len(vs) % 2 else [])
    return vs[0]


def _ffs(mask):
    f = plsc.all_reduce_ffs(mask)
    if f.ndim == 0:
        f = jnp.full((16,), f, jnp.int32)
    return f


def _build_topk():
    info = plsc.get_sparse_core_info()
    nc, ns = info.num_cores, info.num_subcores
    nw = nc * ns
    rows_per = _B // nw
    mesh = plsc.VectorSubcoreMesh(core_axis_name="c", subcore_axis_name="s")

    @functools.partial(
        pl.kernel,
        out_type=[
            jax.ShapeDtypeStruct((_B * _KP,), jnp.float32),
            jax.ShapeDtypeStruct((_B * _KP,), jnp.int32),
        ],
        mesh=mesh,
        scratch_types=[
            pltpu.VMEM((_VP,), jnp.float32),
            pltpu.VMEM((_NBP * 16,), jnp.float32),
            pltpu.VMEM((_NSB * 16,), jnp.float32),
            pltpu.VMEM((_KP,), jnp.float32),
            pltpu.VMEM((_KP,), jnp.int32),
            pltpu.SemaphoreType.DMA,
            pltpu.SemaphoreType.DMA,
            pltpu.SemaphoreType.DMA,
            pltpu.SemaphoreType.DMA,
        ],
        compiler_params=pltpu.CompilerParams(needs_layout_passes=False),
    )
    def topk(logits_hbm, vals_hbm, idx_hbm, row_v, lm_v, l2_v, ov_v, oi_v,
             sem0, sem1, sem2, sem3):
        wid = lax.axis_index("s") * nc + lax.axis_index("c")
        lanes = lax.iota(jnp.int32, 16)
        neg = _splat_f(_NEG)
        bigi = _splat_i(1 << 30)

        for t in range(_NB * 16, _NBP * 16, 16):
            lm_v[pl.ds(t, 16)] = neg

        def row_body(r, carry):
            row = wid * rows_per + r
            base = row * _V
            qb = 98
            q = qb * _BLK
            sems = (sem0, sem1, sem2, sem3)
            copies = []
            for i in range(4):
                lo = i * q
                hi = min((i + 1) * q, _V)
                copies.append(pltpu.async_copy(
                    logits_hbm.at[pl.ds(base + lo, hi - lo)],
                    row_v.at[pl.ds(lo, hi - lo)], sems[i]))

            def _blk(j):
                lm_v[pl.ds(j * 16, 16)] = _tree_max(
                    [row_v[pl.ds(j * _BLK + ii * 16, 16)] for ii in range(16)])

            for i in range(4):
                copies[i].wait()
                if i == 3:
                    for t in range(_V, _VP, 16):
                        row_v[pl.ds(t, 16)] = neg
                hi = _NB if i == 3 else qb * (i + 1)
                plsc.parallel_loop(qb * i, hi, unroll=4)(_blk)

            def sb_body(s, c):
                l2_v[pl.ds(s * 16, 16)] = _tree_max(
                    [lm_v[pl.ds(s * 256 + jj * 16, 16)] for jj in range(16)])
                return c

            lax.fori_loop(0, _NSB, sb_body, 0)

            for t in range(0, _KP, 16):
                ov_v[pl.ds(t, 16)] = neg
                oi_v[pl.ds(t, 16)] = _splat_i(0)

            def k_body(k, c):
                g = _tree_max([l2_v[pl.ds(s * 16, 16)] for s in range(_NSB)])
                m = jnp.max(g)
                mv = jnp.full((16,), m, jnp.float32)

                sv = []
                for s in range(_NSB):
                    f = _ffs(l2_v[pl.ds(s * 16, 16)] == mv)
                    sv.append(jnp.where(f < 16, _splat_i(s), bigi))
                sstar = jnp.min(_tree_min(sv))

                bv = []
                for jj in range(16):
                    f = _ffs(lm_v[pl.ds(sstar * 256 + jj * 16, 16)] == mv)
                    bv.append(jnp.where(f < 16, _splat_i(jj), bigi))
                bstar = sstar * 16 + jnp.min(_tree_min(bv))

                pv = []
                for ii in range(16):
                    f = _ffs(row_v[pl.ds(bstar * _BLK + ii * 16, 16)] == mv)
                    pv.append(jnp.where(f < 16, f + ii * 16, bigi))
                pos = jnp.min(_tree_min(pv))
                flat = bstar * _BLK + pos

                q = (k // 16) * 16
                rl = k % 16
                ov_v[pl.ds(q, 16)] = jnp.where(lanes == rl, mv,
                                               ov_v[pl.ds(q, 16)])
                oi_v[pl.ds(q, 16)] = jnp.where(lanes == rl,
                                               jnp.full((16,), flat, jnp.int32),
                                               oi_v[pl.ds(q, 16)])

                base = bstar * _BLK + (pos // 16) * 16
                row_v[pl.ds(base, 16)] = jnp.where(lanes == pos % 16, neg,
                                                   row_v[pl.ds(base, 16)])

                lm_v[pl.ds(bstar * 16, 16)] = _tree_max(
                    [row_v[pl.ds(bstar * _BLK + ii * 16, 16)]
                     for ii in range(16)])
                l2_v[pl.ds(sstar * 16, 16)] = _tree_max(
                    [lm_v[pl.ds(sstar * 256 + jj * 16, 16)]
                     for jj in range(16)])
                return c

            lax.fori_loop(0, _K, k_body, 0)

            pltpu.sync_copy(ov_v, vals_hbm.at[pl.ds(row * _KP, _KP)])
            pltpu.sync_copy(oi_v, idx_hbm.at[pl.ds(row * _KP, _KP)])
            return carry

        lax.fori_loop(0, rows_per, row_body, 0)

    return topk


def _sample_body(vals_ref, idx_ref, u_ref, out_ref):
    v = vals_ref[...]
    lane = lax.broadcasted_iota(jnp.int32, (_B, _KP), 1)
    valid = lane < _K
    vm = jnp.where(valid, v, -jnp.inf)
    rmax = jnp.max(vm, axis=1, keepdims=True)
    e = jnp.where(valid, jnp.exp(vm - rmax), 0.0)
    p = e / jnp.sum(e, axis=1, keepdims=True)
    g = -jnp.log(-jnp.log(u_ref[...]))
    score = jnp.where(valid, jnp.log(p + 1e-12) + g, -jnp.inf)
    smax = jnp.max(score, axis=1, keepdims=True)
    choice = jnp.min(jnp.where(score == smax, lane, _KP), axis=1, keepdims=True)
    token = jnp.sum(jnp.where(lane == choice, idx_ref[...], 0), axis=1,
                    keepdims=True)
    out_ref[...] = token


def kernel(logits, u):
    vals_f, idx_f = _build_topk()(logits.reshape(-1))
    vals = vals_f.reshape(_B, _KP)
    idx = idx_f.reshape(_B, _KP)
    up = jnp.pad(u, ((0, 0), (0, _KP - _K)), constant_values=0.5)
    token = pl.pallas_call(
        _sample_body,
        out_shape=jax.ShapeDtypeStruct((_B, 1), jnp.int32),
    )(vals, idx, up)
    return token.reshape(_B)
